# Initial kernel scaffold; baseline (speedup 1.0000x reference)
import jax
import jax.numpy as jnp
from jax.experimental import pallas as pl


def kernel(x, edge_index, edge_attr, u, batch, W1, b1, W2, b2):
    raise NotImplementedError("write your pallas kernel here")


# speedup vs baseline: 3.2891x; 3.2891x over previous
import functools

import jax
import jax.numpy as jnp
from jax import lax
from jax.experimental import pallas as pl
from jax.experimental.pallas import tpu as pltpu
from jax.experimental.pallas import tpu_sc as plsc

N_NODES = 10000
D_EDGE = 16
NC = 2
NS = 16
LANES = 16

ROWS_PER_STREAM = 128
STREAMS_PER_GROUP = 16
GROUP_ROWS = ROWS_PER_STREAM * STREAMS_PER_GROUP
GROUPS_PER_TILE = 5
CHUNKS_PER_TILE = STREAMS_PER_GROUP * GROUPS_PER_TILE
E_PAD = NC * NS * CHUNKS_PER_TILE * ROWS_PER_STREAM
DUMP_TILES = 10
DUMP_ROWS = N_NODES // DUMP_TILES


def _sc_body(ea_hbm, col_hbm, out_hbm, idx_v, ea_v, zbuf, shared):
    c = lax.axis_index("c")
    s = lax.axis_index("s")

    @pl.when(s < DUMP_TILES)
    def _():
        def _zero(i, _):
            zbuf[i, :] = jnp.zeros((LANES,), jnp.float32)
            return 0

        lax.fori_loop(0, DUMP_ROWS, _zero, 0)
        pltpu.sync_copy(zbuf, shared.at[pl.ds(s * DUMP_ROWS, DUMP_ROWS)])

    plsc.subcore_barrier()

    tile_chunk0 = (c * NS + s) * CHUNKS_PER_TILE

    def _group(g, _):
        ck = tile_chunk0 + g * STREAMS_PER_GROUP
        pltpu.sync_copy(col_hbm.at[pl.ds(ck, STREAMS_PER_GROUP)], idx_v)
        pltpu.sync_copy(ea_hbm.at[pl.ds(ck * ROWS_PER_STREAM, GROUP_ROWS)], ea_v)
        for j in range(STREAMS_PER_GROUP):
            pltpu.sync_copy(
                ea_v.at[pl.ds(j * ROWS_PER_STREAM, ROWS_PER_STREAM)],
                shared.at[idx_v.at[j]],
                add=True,
            )
        return 0

    lax.fori_loop(0, GROUPS_PER_TILE, _group, 0)
    plsc.subcore_barrier()

    @pl.when(s < DUMP_TILES)
    def _():
        pltpu.sync_copy(
            shared.at[pl.ds(s * DUMP_ROWS, DUMP_ROWS)],
            out_hbm.at[c, pl.ds(s * DUMP_ROWS, DUMP_ROWS)],
        )


@jax.jit
def _sc_scatter_add(ea_pad, col2d):
    mesh = plsc.VectorSubcoreMesh(core_axis_name="c", subcore_axis_name="s")
    return pl.kernel(
        _sc_body,
        out_type=jax.ShapeDtypeStruct((NC, N_NODES, D_EDGE), jnp.float32),
        mesh=mesh,
        scratch_types=[
            pltpu.VMEM((STREAMS_PER_GROUP, ROWS_PER_STREAM), jnp.int32),
            pltpu.VMEM((GROUP_ROWS, D_EDGE), jnp.float32),
            pltpu.VMEM((DUMP_ROWS, D_EDGE), jnp.float32),
            pltpu.VMEM_SHARED((N_NODES, D_EDGE), jnp.float32),
        ],
        compiler_params=pltpu.CompilerParams(use_tc_tiling_on_sc=False),
    )(ea_pad, col2d)


def _tc_body(x_ref, parts_ref, batch_ref, u_ref, w1x_ref, w1e_ref, w1u_ref,
             b1_ref, w2_ref, b2_ref, out_ref):
    B = x_ref.shape[0]
    agg = parts_ref[0] + parts_ref[1]
    g = jnp.dot(u_ref[...], w1u_ref[...], preferred_element_type=jnp.float32)
    onehot = (batch_ref[...] == lax.broadcasted_iota(jnp.int32, (B, 16), 1)
              ).astype(jnp.float32)
    pre = (jnp.dot(x_ref[...], w1x_ref[...], preferred_element_type=jnp.float32)
           + jnp.dot(agg, w1e_ref[...], preferred_element_type=jnp.float32)
           + jnp.dot(onehot, g, preferred_element_type=jnp.float32)
           + b1_ref[...])
    h = pre * jax.nn.sigmoid(pre)
    pre2 = jnp.dot(h, w2_ref[...], preferred_element_type=jnp.float32) + b2_ref[...]
    out_ref[...] = pre2 * jax.nn.sigmoid(pre2)


@functools.partial(jax.jit, static_argnames=("block",))
def _tc_mlp(x, parts, batch2d, u, w1x, w1e, w1u, b1, w2, b2, block=1000):
    nblk = N_NODES // block
    k = w1u.shape[1]
    return pl.pallas_call(
        _tc_body,
        grid=(nblk,),
        in_specs=[
            pl.BlockSpec((block, x.shape[1]), lambda i: (i, 0)),
            pl.BlockSpec((NC, block, D_EDGE), lambda i: (0, i, 0)),
            pl.BlockSpec((block, 1), lambda i: (i, 0)),
            pl.BlockSpec(u.shape, lambda i: (0, 0)),
            pl.BlockSpec(w1x.shape, lambda i: (0, 0)),
            pl.BlockSpec(w1e.shape, lambda i: (0, 0)),
            pl.BlockSpec(w1u.shape, lambda i: (0, 0)),
            pl.BlockSpec(b1.shape, lambda i: (0, 0)),
            pl.BlockSpec(w2.shape, lambda i: (0, 0)),
            pl.BlockSpec(b2.shape, lambda i: (0, 0)),
        ],
        out_specs=pl.BlockSpec((block, k), lambda i: (i, 0)),
        out_shape=jax.ShapeDtypeStruct((N_NODES, k), jnp.float32),
    )(x, parts, batch2d, u, w1x, w1e, w1u, b1, w2, b2)


def kernel(x, edge_index, edge_attr, u, batch, W1, b1, W2, b2):
    col = edge_index[1].astype(jnp.int32)
    e = col.shape[0]
    pad = E_PAD - e
    col2d = jnp.concatenate([col, jnp.zeros((pad,), jnp.int32)]).reshape(
        E_PAD // ROWS_PER_STREAM, ROWS_PER_STREAM)
    ea_pad = jnp.concatenate(
        [edge_attr, jnp.zeros((pad, D_EDGE), edge_attr.dtype)], axis=0)

    parts = _sc_scatter_add(ea_pad, col2d)

    d_feat = x.shape[1]
    u_dim = u.shape[1]
    w1x = W1[:d_feat]
    w1e = W1[d_feat:d_feat + D_EDGE]
    w1u = W1[d_feat + D_EDGE:]
    batch2d = batch.astype(jnp.int32).reshape(-1, 1)
    return _tc_mlp(x, parts, batch2d, u, w1x, w1e, w1u,
                   b1.reshape(1, -1), W2, b2.reshape(1, -1))


# speedup vs baseline: 5.1990x; 1.5806x over previous
import functools

import jax
import jax.numpy as jnp
from jax import lax
from jax.experimental import pallas as pl
from jax.experimental.pallas import tpu as pltpu
from jax.experimental.pallas import tpu_sc as plsc

N_NODES = 10000
D_EDGE = 16
NC = 2
NS = 16
LANES = 16

N_EDGES = 320000
ROWS_PER_STREAM = 128
N_CHUNKS = N_EDGES // ROWS_PER_STREAM
STREAMS_PER_GROUP = 20
GROUP_ROWS = ROWS_PER_STREAM * STREAMS_PER_GROUP
N_GROUPS = N_CHUNKS // STREAMS_PER_GROUP
FULL_TILES = 29
DUMP_TILES = 10
DUMP_ROWS = N_NODES // DUMP_TILES


def _sc_body(ea_hbm, col_hbm, out_hbm, idx_v, ea_v, zbuf, shared, sem):
    c = lax.axis_index("c")
    s = lax.axis_index("s")

    @pl.when(s < DUMP_TILES)
    def _():
        def _zero(i, _):
            zbuf[i, :] = jnp.zeros((LANES,), jnp.float32)
            return 0

        lax.fori_loop(0, DUMP_ROWS, _zero, 0)
        pltpu.sync_copy(zbuf, shared.at[pl.ds(s * DUMP_ROWS, DUMP_ROWS)])

    plsc.subcore_barrier()

    t = c * NS + s
    gstart = jnp.where(t < FULL_TILES, 4 * t, 4 * FULL_TILES + 3 * (t - FULL_TILES))
    ngroups = jnp.where(t < FULL_TILES, 4, 3)

    def _group(g, _):
        ck = (gstart + g) * STREAMS_PER_GROUP
        pltpu.sync_copy(col_hbm.at[pl.ds(ck, STREAMS_PER_GROUP)], idx_v)
        pltpu.sync_copy(ea_hbm.at[pl.ds(ck * ROWS_PER_STREAM, GROUP_ROWS)], ea_v)
        handles = [
            pltpu.async_copy(
                ea_v.at[pl.ds(j * ROWS_PER_STREAM, ROWS_PER_STREAM)],
                shared.at[idx_v.at[j]],
                sem,
                add=True,
            )
            for j in range(STREAMS_PER_GROUP)
        ]
        for h in handles:
            h.wait()
        return 0

    lax.fori_loop(0, ngroups, _group, 0)
    plsc.subcore_barrier()

    @pl.when(s < DUMP_TILES)
    def _():
        pltpu.sync_copy(
            shared.at[pl.ds(s * DUMP_ROWS, DUMP_ROWS)],
            out_hbm.at[c, pl.ds(s * DUMP_ROWS, DUMP_ROWS)],
        )


@jax.jit
def _sc_scatter_add(ea_pad, col2d):
    mesh = plsc.VectorSubcoreMesh(core_axis_name="c", subcore_axis_name="s")
    return pl.kernel(
        _sc_body,
        out_type=jax.ShapeDtypeStruct((NC, N_NODES, D_EDGE), jnp.float32),
        mesh=mesh,
        scratch_types=[
            pltpu.VMEM((STREAMS_PER_GROUP, ROWS_PER_STREAM), jnp.int32),
            pltpu.VMEM((GROUP_ROWS, D_EDGE), jnp.float32),
            pltpu.VMEM((DUMP_ROWS, D_EDGE), jnp.float32),
            pltpu.VMEM_SHARED((N_NODES, D_EDGE), jnp.float32),
            pltpu.SemaphoreType.DMA,
        ],
        compiler_params=pltpu.CompilerParams(use_tc_tiling_on_sc=False),
    )(ea_pad, col2d)


def _tc_body(x_ref, parts_ref, batch_ref, u_ref, w1x_ref, w1e_ref, w1u_ref,
             b1_ref, w2_ref, b2_ref, out_ref):
    B = x_ref.shape[0]
    agg = parts_ref[0] + parts_ref[1]
    g = jnp.dot(u_ref[...], w1u_ref[...], preferred_element_type=jnp.float32)
    onehot = (batch_ref[...] == lax.broadcasted_iota(jnp.int32, (B, 16), 1)
              ).astype(jnp.float32)
    pre = (jnp.dot(x_ref[...], w1x_ref[...], preferred_element_type=jnp.float32)
           + jnp.dot(agg, w1e_ref[...], preferred_element_type=jnp.float32)
           + jnp.dot(onehot, g, preferred_element_type=jnp.float32)
           + b1_ref[...])
    h = pre * jax.nn.sigmoid(pre)
    pre2 = jnp.dot(h, w2_ref[...], preferred_element_type=jnp.float32) + b2_ref[...]
    out_ref[...] = pre2 * jax.nn.sigmoid(pre2)


@functools.partial(jax.jit, static_argnames=("block",))
def _tc_mlp(x, parts, batch2d, u, w1x, w1e, w1u, b1, w2, b2, block=1000):
    nblk = N_NODES // block
    k = w1u.shape[1]
    return pl.pallas_call(
        _tc_body,
        grid=(nblk,),
        in_specs=[
            pl.BlockSpec((block, x.shape[1]), lambda i: (i, 0)),
            pl.BlockSpec((NC, block, D_EDGE), lambda i: (0, i, 0)),
            pl.BlockSpec((block, 1), lambda i: (i, 0)),
            pl.BlockSpec(u.shape, lambda i: (0, 0)),
            pl.BlockSpec(w1x.shape, lambda i: (0, 0)),
            pl.BlockSpec(w1e.shape, lambda i: (0, 0)),
            pl.BlockSpec(w1u.shape, lambda i: (0, 0)),
            pl.BlockSpec(b1.shape, lambda i: (0, 0)),
            pl.BlockSpec(w2.shape, lambda i: (0, 0)),
            pl.BlockSpec(b2.shape, lambda i: (0, 0)),
        ],
        out_specs=pl.BlockSpec((block, k), lambda i: (i, 0)),
        out_shape=jax.ShapeDtypeStruct((N_NODES, k), jnp.float32),
    )(x, parts, batch2d, u, w1x, w1e, w1u, b1, w2, b2)


def kernel(x, edge_index, edge_attr, u, batch, W1, b1, W2, b2):
    col2d = edge_index[1].astype(jnp.int32).reshape(N_CHUNKS, ROWS_PER_STREAM)
    parts = _sc_scatter_add(edge_attr, col2d)

    d_feat = x.shape[1]
    u_dim = u.shape[1]
    w1x = W1[:d_feat]
    w1e = W1[d_feat:d_feat + D_EDGE]
    w1u = W1[d_feat + D_EDGE:]
    batch2d = batch.astype(jnp.int32).reshape(-1, 1)
    return _tc_mlp(x, parts, batch2d, u, w1x, w1e, w1u,
                   b1.reshape(1, -1), W2, b2.reshape(1, -1))
